# baseline (speedup 1.0000x reference)
import jax
import jax.numpy as jnp
from jax.experimental import pallas as pl
from jax.experimental.pallas import tpu as pltpu

N, F_IN, H, C = 10000, 128, 128, 64
BR = 400
W = 1024
NR = N // BR
NCB = -(-N // W)
NP = NCB * W


def _sage_kernel(adj_ref, x_ref, wl1_ref, bl1_ref, wr1_ref,
                 wl2_ref, bl2_ref, wr2_ref, out_ref,
                 hw_ref, hr_ref, oacc_ref, agg_ref, diag_ref):
    p = pl.program_id(0)
    r = pl.program_id(1)
    cb = pl.program_id(2)

    @pl.when(p == 0)
    def _phase_a():
        a16 = adj_ref[...].astype(jnp.bfloat16)
        x16 = x_ref[pl.ds(cb * W, W), :].astype(jnp.bfloat16)

        @pl.when((p == 0) & (r == 0) & (cb == 0))
        def _():
            hw_ref[pl.ds(N, NP - N), :] = jnp.zeros((NP - N, C), jnp.bfloat16)

        part = jnp.dot(a16, x16, preferred_element_type=jnp.float32)

        @pl.when(cb == 0)
        def _():
            oacc_ref[pl.ds(r * BR, BR), :] = jnp.zeros((BR, C), jnp.float32)
            agg_ref[...] = part

        @pl.when(cb > 0)
        def _():
            agg_ref[...] += part

        @pl.when(W * (cb + 1) <= BR * r)
        def _():
            hw = hw_ref[pl.ds(cb * W, W), :]
            oacc_ref[pl.ds(r * BR, BR), :] += jnp.dot(
                a16, hw, preferred_element_type=jnp.float32)

        @pl.when((W * (cb + 1) > BR * r) & (W * (cb + 1) <= BR * (r + 1)))
        def _():
            diag_ref[...] = a16

        @pl.when(cb == NCB - 1)
        def _():
            o = jax.lax.dot_general(agg_ref[...], wl1_ref[...],
                                    (((1,), (1,)), ((), ())),
                                    preferred_element_type=jnp.float32)
            o = o + bl1_ref[...]
            o = o + jax.lax.dot_general(x_ref[pl.ds(r * BR, BR), :],
                                        wr1_ref[...],
                                        (((1,), (1,)), ((), ())),
                                        preferred_element_type=jnp.float32)
            denom = jnp.maximum(jnp.sum(jnp.abs(o), axis=1, keepdims=True),
                                1e-12)
            h = jnp.maximum(o / denom, 0.0)
            hw_r = jax.lax.dot_general(h, wl2_ref[...],
                                       (((1,), (1,)), ((), ())),
                                       preferred_element_type=jnp.float32
                                       ).astype(jnp.bfloat16)
            hw_ref[pl.ds(r * BR, BR), :] = hw_r
            hr_ref[pl.ds(r * BR, BR), :] = jax.lax.dot_general(
                h, wr2_ref[...], (((1,), (1,)), ((), ())),
                preferred_element_type=jnp.float32) + bl2_ref[...]

            q0 = (BR * r) // W
            q1 = (BR * (r + 1)) // W

            @pl.when(q1 > q0)
            def _():
                hw_d = hw_ref[pl.ds((q1 - 1) * W, W), :]
                oacc_ref[pl.ds(r * BR, BR), :] += jnp.dot(
                    diag_ref[...], hw_d, preferred_element_type=jnp.float32)

    @pl.when(p == 1)
    def _phase_b():
        q1 = (BR * (r + 1)) // W

        @pl.when(cb >= q1)
        def _():
            a16 = adj_ref[...].astype(jnp.bfloat16)
            hw = hw_ref[pl.ds(cb * W, W), :]
            oacc_ref[pl.ds(r * BR, BR), :] += jnp.dot(
                a16, hw, preferred_element_type=jnp.float32)

        @pl.when(cb == NCB - 1)
        def _():
            o = oacc_ref[pl.ds(r * BR, BR), :] + hr_ref[pl.ds(r * BR, BR), :]
            m = jnp.max(o, axis=1, keepdims=True)
            lse = jnp.log(jnp.sum(jnp.exp(o - m), axis=1, keepdims=True))
            out_ref[...] = o - m - lse


@jax.jit
def kernel(x, adjs, W_l1, b_l1, W_r1, W_l2, b_l2, W_r2):
    bl1 = b_l1.reshape(1, H)
    bl2 = b_l2.reshape(1, C)
    xp = jnp.zeros((NP, F_IN), jnp.float32).at[:N].set(x)

    def adj_index(p, r, cb):
        q1 = (BR * (r + 1)) // W
        cb_b = jnp.minimum(jnp.maximum(cb, q1), NCB - 1)
        return (r, jnp.where(p == 0, cb, cb_b))

    return pl.pallas_call(
        _sage_kernel,
        grid=(2, NR, NCB),
        in_specs=[
            pl.BlockSpec((BR, W), adj_index),
            pl.BlockSpec((NP, F_IN), lambda p, r, cb: (0, 0)),
            pl.BlockSpec((H, F_IN), lambda p, r, cb: (0, 0)),
            pl.BlockSpec((1, H), lambda p, r, cb: (0, 0)),
            pl.BlockSpec((H, F_IN), lambda p, r, cb: (0, 0)),
            pl.BlockSpec((C, H), lambda p, r, cb: (0, 0)),
            pl.BlockSpec((1, C), lambda p, r, cb: (0, 0)),
            pl.BlockSpec((C, H), lambda p, r, cb: (0, 0)),
        ],
        out_specs=pl.BlockSpec((BR, C), lambda p, r, cb: (r, 0)),
        out_shape=jax.ShapeDtypeStruct((N, C), jnp.float32),
        scratch_shapes=[
            pltpu.VMEM((NP, C), jnp.bfloat16),
            pltpu.VMEM((N, C), jnp.float32),
            pltpu.VMEM((N, C), jnp.float32),
            pltpu.VMEM((BR, F_IN), jnp.float32),
            pltpu.VMEM((BR, W), jnp.bfloat16),
        ],
    )(adjs, xp, W_l1, bl1, W_r1, W_l2, bl2, W_r2)


# speedup vs baseline: 1.2645x; 1.2645x over previous
import jax
import jax.numpy as jnp
from jax.experimental import pallas as pl

N, F_IN, H, C = 10000, 128, 128, 64
BLK = 400
W2 = 2048
NB2 = -(-N // W2)
NP2 = NB2 * W2


def _phase_a(adj_ref, xf_ref, xb_ref, wl1_ref, bl1_ref, wr1_ref,
             wl2_ref, bl2_ref, wr2_ref, hw_ref, acc_ref):
    r = pl.program_id(0)
    a16 = adj_ref[...].astype(jnp.bfloat16)
    agg = jnp.dot(a16, xf_ref[...].astype(jnp.bfloat16),
                  preferred_element_type=jnp.float32)
    o = jax.lax.dot_general(agg, wl1_ref[...], (((1,), (1,)), ((), ())),
                            preferred_element_type=jnp.float32)
    o = o + bl1_ref[...]
    o = o + jax.lax.dot_general(xb_ref[...], wr1_ref[...],
                                (((1,), (1,)), ((), ())),
                                preferred_element_type=jnp.float32)
    denom = jnp.maximum(jnp.sum(jnp.abs(o), axis=1, keepdims=True), 1e-12)
    h = jnp.maximum(o / denom, 0.0)

    @pl.when(r == 0)
    def _():
        hw_ref[...] = jnp.zeros((N, C), jnp.bfloat16)

    hw_ref[pl.ds(r * BLK, BLK), :] = jax.lax.dot_general(
        h, wl2_ref[...], (((1,), (1,)), ((), ())),
        preferred_element_type=jnp.float32).astype(jnp.bfloat16)

    acc = jnp.dot(a16, hw_ref[...], preferred_element_type=jnp.float32)
    acc_ref[pl.ds(r * BLK, BLK), :] = acc + jax.lax.dot_general(
        h, wr2_ref[...], (((1,), (1,)), ((), ())),
        preferred_element_type=jnp.float32) + bl2_ref[...]


def _phase_b(adj_ref, hwp_ref, acc_ref, out_ref):
    i = pl.program_id(0)
    j = pl.program_id(1)
    jstart = (BLK * (i + 1)) // W2

    @pl.when(j == jstart)
    def _():
        out_ref[...] = acc_ref[...]

    @pl.when(j >= jstart)
    def _():
        start_off = BLK * (i + 1) - j * W2
        end_off = N - j * W2
        col = jax.lax.broadcasted_iota(jnp.int32, (BLK, W2), 1)
        a = jnp.where((col >= start_off) & (col < end_off),
                      adj_ref[...], 0.0).astype(jnp.bfloat16)
        out_ref[...] += jnp.dot(a, hwp_ref[pl.ds(j * W2, W2), :],
                                preferred_element_type=jnp.float32)

    @pl.when(j == NB2 - 1)
    def _():
        o = out_ref[...]
        m = jnp.max(o, axis=1, keepdims=True)
        lse = jnp.log(jnp.sum(jnp.exp(o - m), axis=1, keepdims=True))
        out_ref[...] = o - m - lse


@jax.jit
def kernel(x, adjs, W_l1, b_l1, W_r1, W_l2, b_l2, W_r2):
    nblk = N // BLK
    bl1 = b_l1.reshape(1, H)
    bl2 = b_l2.reshape(1, C)

    hw, acc = pl.pallas_call(
        _phase_a,
        grid=(nblk,),
        in_specs=[
            pl.BlockSpec((BLK, N), lambda r: (r, 0)),
            pl.BlockSpec((N, F_IN), lambda r: (0, 0)),
            pl.BlockSpec((BLK, F_IN), lambda r: (r, 0)),
            pl.BlockSpec((H, F_IN), lambda r: (0, 0)),
            pl.BlockSpec((1, H), lambda r: (0, 0)),
            pl.BlockSpec((H, F_IN), lambda r: (0, 0)),
            pl.BlockSpec((C, H), lambda r: (0, 0)),
            pl.BlockSpec((1, C), lambda r: (0, 0)),
            pl.BlockSpec((C, H), lambda r: (0, 0)),
        ],
        out_specs=[
            pl.BlockSpec((N, C), lambda r: (0, 0)),
            pl.BlockSpec((N, C), lambda r: (0, 0)),
        ],
        out_shape=[
            jax.ShapeDtypeStruct((N, C), jnp.bfloat16),
            jax.ShapeDtypeStruct((N, C), jnp.float32),
        ],
    )(adjs, x, x, W_l1, bl1, W_r1, W_l2, bl2, W_r2)

    hwp = jnp.zeros((NP2, C), jnp.bfloat16).at[:N].set(hw)

    def adj_b_index(i, j):
        return (i, jnp.maximum(j, (BLK * (i + 1)) // W2))

    return pl.pallas_call(
        _phase_b,
        grid=(nblk, NB2),
        in_specs=[
            pl.BlockSpec((BLK, W2), adj_b_index),
            pl.BlockSpec((NP2, C), lambda i, j: (0, 0)),
            pl.BlockSpec((BLK, C), lambda i, j: (i, 0)),
        ],
        out_specs=pl.BlockSpec((BLK, C), lambda i, j: (i, 0)),
        out_shape=jax.ShapeDtypeStruct((N, C), jnp.float32),
    )(adjs, hwp, acc)


# speedup vs baseline: 1.8714x; 1.4799x over previous
import jax
import jax.numpy as jnp
from jax.experimental import pallas as pl
from jax.experimental.pallas import tpu as pltpu

N, F_IN, H, C = 10000, 128, 128, 64
BLK = 400
W2 = 5120
NB2 = 2
NP2 = NB2 * W2


def _phase_a(adj_ref, xf_ref, xb_ref, wl1_ref, bl1_ref, wr1_ref,
             wl2_ref, bl2_ref, wr2_ref, hw_ref, acc_ref, rhs_ref):
    r = pl.program_id(0)

    @pl.when(r == 0)
    def _():
        rhs_ref[:, :F_IN] = xf_ref[...].astype(jnp.bfloat16)
        rhs_ref[:, F_IN:] = jnp.zeros((N, C), jnp.bfloat16)

    a16 = adj_ref[...].astype(jnp.bfloat16)
    big = jnp.dot(a16, rhs_ref[...], preferred_element_type=jnp.float32)
    agg = big[:, :F_IN]
    pfx = big[:, F_IN:]

    o = jax.lax.dot_general(agg, wl1_ref[...], (((1,), (1,)), ((), ())),
                            preferred_element_type=jnp.float32)
    o = o + bl1_ref[...]
    o = o + jax.lax.dot_general(xb_ref[...], wr1_ref[...],
                                (((1,), (1,)), ((), ())),
                                preferred_element_type=jnp.float32)
    denom = jnp.maximum(jnp.sum(jnp.abs(o), axis=1, keepdims=True), 1e-12)
    h = jnp.maximum(o / denom, 0.0)

    hw_r = jax.lax.dot_general(
        h, wl2_ref[...], (((1,), (1,)), ((), ())),
        preferred_element_type=jnp.float32).astype(jnp.bfloat16)
    rhs_ref[pl.ds(r * BLK, BLK), F_IN:] = hw_r
    hw_ref[pl.ds(r * BLK, BLK), :] = hw_r
    acc_ref[pl.ds(r * BLK, BLK), :] = pfx + jax.lax.dot_general(
        h, wr2_ref[...], (((1,), (1,)), ((), ())),
        preferred_element_type=jnp.float32) + bl2_ref[...]


def _phase_b(adj_ref, hwp_ref, acc_ref, out_ref):
    i = pl.program_id(0)
    j = pl.program_id(1)
    jstart = (BLK * i) // W2

    @pl.when(j == jstart)
    def _():
        out_ref[...] = acc_ref[...]

    @pl.when(j >= jstart)
    def _():
        start_off = BLK * i - j * W2
        end_off = N - j * W2
        col = jax.lax.broadcasted_iota(jnp.int32, (BLK, W2), 1)
        a = jnp.where((col >= start_off) & (col < end_off),
                      adj_ref[...], 0.0).astype(jnp.bfloat16)
        out_ref[...] += jnp.dot(a, hwp_ref[pl.ds(j * W2, W2), :],
                                preferred_element_type=jnp.float32)

    @pl.when(j == NB2 - 1)
    def _():
        o = out_ref[...]
        m = jnp.max(o, axis=1, keepdims=True)
        lse = jnp.log(jnp.sum(jnp.exp(o - m), axis=1, keepdims=True))
        out_ref[...] = o - m - lse


@jax.jit
def kernel(x, adjs, W_l1, b_l1, W_r1, W_l2, b_l2, W_r2):
    nblk = N // BLK
    bl1 = b_l1.reshape(1, H)
    bl2 = b_l2.reshape(1, C)

    hw, acc = pl.pallas_call(
        _phase_a,
        grid=(nblk,),
        in_specs=[
            pl.BlockSpec((BLK, N), lambda r: (r, 0)),
            pl.BlockSpec((N, F_IN), lambda r: (0, 0)),
            pl.BlockSpec((BLK, F_IN), lambda r: (r, 0)),
            pl.BlockSpec((H, F_IN), lambda r: (0, 0)),
            pl.BlockSpec((1, H), lambda r: (0, 0)),
            pl.BlockSpec((H, F_IN), lambda r: (0, 0)),
            pl.BlockSpec((C, H), lambda r: (0, 0)),
            pl.BlockSpec((1, C), lambda r: (0, 0)),
            pl.BlockSpec((C, H), lambda r: (0, 0)),
        ],
        out_specs=[
            pl.BlockSpec((N, C), lambda r: (0, 0)),
            pl.BlockSpec((N, C), lambda r: (0, 0)),
        ],
        out_shape=[
            jax.ShapeDtypeStruct((N, C), jnp.bfloat16),
            jax.ShapeDtypeStruct((N, C), jnp.float32),
        ],
        scratch_shapes=[
            pltpu.VMEM((N, F_IN + C), jnp.bfloat16),
        ],
    )(adjs, x, x, W_l1, bl1, W_r1, W_l2, bl2, W_r2)

    hwp = jnp.zeros((NP2, C), jnp.bfloat16).at[:N].set(hw)

    def adj_b_index(i, j):
        return (i, jnp.maximum(j, (BLK * i) // W2))

    return pl.pallas_call(
        _phase_b,
        grid=(nblk, NB2),
        in_specs=[
            pl.BlockSpec((BLK, W2), adj_b_index),
            pl.BlockSpec((NP2, C), lambda i, j: (0, 0)),
            pl.BlockSpec((BLK, C), lambda i, j: (i, 0)),
        ],
        out_specs=pl.BlockSpec((BLK, C), lambda i, j: (i, 0)),
        out_shape=jax.ShapeDtypeStruct((N, C), jnp.float32),
    )(adjs, hwp, acc)


# speedup vs baseline: 1.8880x; 1.0089x over previous
import jax
import jax.numpy as jnp
from jax.experimental import pallas as pl
from jax.experimental.pallas import tpu as pltpu

N, F_IN, H, C = 10000, 128, 128, 64
BLK = 400
BLKB = 1000
W2 = 5120
NB2 = 2
NP2 = NB2 * W2


def _phase_a(adj_ref, xf_ref, xb_ref, wl1_ref, bl1_ref, wr1_ref,
             wl2_ref, bl2_ref, wr2_ref, hw_ref, acc_ref, rhs_ref):
    r = pl.program_id(0)

    @pl.when(r == 0)
    def _():
        rhs_ref[:, :F_IN] = xf_ref[...].astype(jnp.bfloat16)
        rhs_ref[:, F_IN:] = jnp.zeros((N, C), jnp.bfloat16)

    a16 = adj_ref[...].astype(jnp.bfloat16)
    big = jnp.dot(a16, rhs_ref[...], preferred_element_type=jnp.float32)
    agg = big[:, :F_IN]
    pfx = big[:, F_IN:]

    o = jax.lax.dot_general(agg, wl1_ref[...], (((1,), (1,)), ((), ())),
                            preferred_element_type=jnp.float32)
    o = o + bl1_ref[...]
    o = o + jax.lax.dot_general(xb_ref[...], wr1_ref[...],
                                (((1,), (1,)), ((), ())),
                                preferred_element_type=jnp.float32)
    denom = jnp.maximum(jnp.sum(jnp.abs(o), axis=1, keepdims=True), 1e-12)
    h = jnp.maximum(o / denom, 0.0)

    hw_r = jax.lax.dot_general(
        h, wl2_ref[...], (((1,), (1,)), ((), ())),
        preferred_element_type=jnp.float32).astype(jnp.bfloat16)
    rhs_ref[pl.ds(r * BLK, BLK), F_IN:] = hw_r
    hw_ref[pl.ds(r * BLK, BLK), :] = hw_r
    acc_ref[pl.ds(r * BLK, BLK), :] = pfx + jax.lax.dot_general(
        h, wr2_ref[...], (((1,), (1,)), ((), ())),
        preferred_element_type=jnp.float32) + bl2_ref[...]


def _phase_b(adj_ref, hwp_ref, acc_ref, out_ref):
    i = pl.program_id(0)
    j = pl.program_id(1)
    jstart = (BLKB * i) // W2

    @pl.when(j == jstart)
    def _():
        out_ref[...] = acc_ref[...]

    @pl.when(j >= jstart)
    def _():
        start_off = BLKB * i - j * W2
        end_off = N - j * W2
        row = jax.lax.broadcasted_iota(jnp.int32, (W2, C), 0)
        hw_t = jnp.where((row >= start_off) & (row < end_off),
                         hwp_ref[pl.ds(j * W2, W2), :], 0)
        out_ref[...] += jnp.dot(adj_ref[...].astype(jnp.bfloat16), hw_t,
                                preferred_element_type=jnp.float32)

    @pl.when(j == NB2 - 1)
    def _():
        o = out_ref[...]
        m = jnp.max(o, axis=1, keepdims=True)
        lse = jnp.log(jnp.sum(jnp.exp(o - m), axis=1, keepdims=True))
        out_ref[...] = o - m - lse


@jax.jit
def kernel(x, adjs, W_l1, b_l1, W_r1, W_l2, b_l2, W_r2):
    nblk = N // BLK
    bl1 = b_l1.reshape(1, H)
    bl2 = b_l2.reshape(1, C)

    hw, acc = pl.pallas_call(
        _phase_a,
        grid=(nblk,),
        in_specs=[
            pl.BlockSpec((BLK, N), lambda r: (r, 0)),
            pl.BlockSpec((N, F_IN), lambda r: (0, 0)),
            pl.BlockSpec((BLK, F_IN), lambda r: (r, 0)),
            pl.BlockSpec((H, F_IN), lambda r: (0, 0)),
            pl.BlockSpec((1, H), lambda r: (0, 0)),
            pl.BlockSpec((H, F_IN), lambda r: (0, 0)),
            pl.BlockSpec((C, H), lambda r: (0, 0)),
            pl.BlockSpec((1, C), lambda r: (0, 0)),
            pl.BlockSpec((C, H), lambda r: (0, 0)),
        ],
        out_specs=[
            pl.BlockSpec((N, C), lambda r: (0, 0)),
            pl.BlockSpec((N, C), lambda r: (0, 0)),
        ],
        out_shape=[
            jax.ShapeDtypeStruct((N, C), jnp.bfloat16),
            jax.ShapeDtypeStruct((N, C), jnp.float32),
        ],
        scratch_shapes=[
            pltpu.VMEM((N, F_IN + C), jnp.bfloat16),
        ],
    )(adjs, x, x, W_l1, bl1, W_r1, W_l2, bl2, W_r2)

    hwp = jnp.zeros((NP2, C), jnp.bfloat16).at[:N].set(hw)

    def adj_b_index(i, j):
        return (i, jnp.maximum(j, (BLKB * i) // W2))

    return pl.pallas_call(
        _phase_b,
        grid=(N // BLKB, NB2),
        in_specs=[
            pl.BlockSpec((BLKB, W2), adj_b_index),
            pl.BlockSpec((NP2, C), lambda i, j: (0, 0)),
            pl.BlockSpec((BLKB, C), lambda i, j: (i, 0)),
        ],
        out_specs=pl.BlockSpec((BLKB, C), lambda i, j: (i, 0)),
        out_shape=jax.ShapeDtypeStruct((N, C), jnp.float32),
    )(adjs, hwp, acc)


# speedup vs baseline: 1.9576x; 1.0368x over previous
import jax
import jax.numpy as jnp
from jax.experimental import pallas as pl
from jax.experimental.pallas import tpu as pltpu

N, F_IN, H, C = 10000, 128, 128, 64
BLK = 400
CHUNK = 2000
BLKB = 2000
W2 = 2560
NB2 = 4
NP2 = NB2 * W2


def _phase_a(adj_ref, xf_ref, xb_ref, wl1_ref, bl1_ref, wr1_ref,
             wl2_ref, bl2_ref, wr2_ref, hw_ref, acc_ref, rhs_ref):
    r = pl.program_id(0)

    @pl.when(r == 0)
    def _():
        rhs_ref[:, :F_IN] = xf_ref[...].astype(jnp.bfloat16)
        rhs_ref[:, F_IN:] = jnp.zeros((N, C), jnp.bfloat16)

    a16 = adj_ref[...].astype(jnp.bfloat16)
    big = jnp.dot(a16, rhs_ref[...], preferred_element_type=jnp.float32)
    agg = big[:, :F_IN]
    pfx = big[:, F_IN:]

    o = jax.lax.dot_general(agg, wl1_ref[...], (((1,), (1,)), ((), ())),
                            preferred_element_type=jnp.float32)
    o = o + bl1_ref[...]
    o = o + jax.lax.dot_general(xb_ref[...], wr1_ref[...],
                                (((1,), (1,)), ((), ())),
                                preferred_element_type=jnp.float32)
    denom = jnp.maximum(jnp.sum(jnp.abs(o), axis=1, keepdims=True), 1e-12)
    h = jnp.maximum(o / denom, 0.0)

    hw_r = jax.lax.dot_general(
        h, wl2_ref[...], (((1,), (1,)), ((), ())),
        preferred_element_type=jnp.float32).astype(jnp.bfloat16)
    hw_ref[pl.ds(r * BLK, BLK), :] = hw_r
    acc_ref[pl.ds(r * BLK, BLK), :] = pfx + jax.lax.dot_general(
        h, wr2_ref[...], (((1,), (1,)), ((), ())),
        preferred_element_type=jnp.float32) + bl2_ref[...]

    @pl.when(((r + 1) * BLK) % CHUNK == 0)
    def _():
        q = (r * BLK) // CHUNK
        rhs_ref[pl.ds(q * CHUNK, CHUNK), F_IN:] = \
            hw_ref[pl.ds(q * CHUNK, CHUNK), :]


def _phase_b(adj_ref, hwp_ref, acc_ref, out_ref):
    i = pl.program_id(0)
    j = pl.program_id(1)
    jstart = (BLKB * i) // W2

    @pl.when(j == jstart)
    def _():
        out_ref[...] = acc_ref[...]

    @pl.when(j >= jstart)
    def _():
        start_off = BLKB * i - j * W2
        end_off = N - j * W2
        row = jax.lax.broadcasted_iota(jnp.int32, (W2, C), 0)
        hw_t = jnp.where((row >= start_off) & (row < end_off),
                         hwp_ref[pl.ds(j * W2, W2), :], 0)
        out_ref[...] += jnp.dot(adj_ref[...].astype(jnp.bfloat16), hw_t,
                                preferred_element_type=jnp.float32)

    @pl.when(j == NB2 - 1)
    def _():
        o = out_ref[...]
        m = jnp.max(o, axis=1, keepdims=True)
        lse = jnp.log(jnp.sum(jnp.exp(o - m), axis=1, keepdims=True))
        out_ref[...] = o - m - lse


@jax.jit
def kernel(x, adjs, W_l1, b_l1, W_r1, W_l2, b_l2, W_r2):
    nblk = N // BLK
    bl1 = b_l1.reshape(1, H)
    bl2 = b_l2.reshape(1, C)

    hw, acc = pl.pallas_call(
        _phase_a,
        grid=(nblk,),
        in_specs=[
            pl.BlockSpec((BLK, N), lambda r: (r, 0)),
            pl.BlockSpec((N, F_IN), lambda r: (0, 0)),
            pl.BlockSpec((BLK, F_IN), lambda r: (r, 0)),
            pl.BlockSpec((H, F_IN), lambda r: (0, 0)),
            pl.BlockSpec((1, H), lambda r: (0, 0)),
            pl.BlockSpec((H, F_IN), lambda r: (0, 0)),
            pl.BlockSpec((C, H), lambda r: (0, 0)),
            pl.BlockSpec((1, C), lambda r: (0, 0)),
            pl.BlockSpec((C, H), lambda r: (0, 0)),
        ],
        out_specs=[
            pl.BlockSpec((N, C), lambda r: (0, 0)),
            pl.BlockSpec((N, C), lambda r: (0, 0)),
        ],
        out_shape=[
            jax.ShapeDtypeStruct((N, C), jnp.bfloat16),
            jax.ShapeDtypeStruct((N, C), jnp.float32),
        ],
        scratch_shapes=[
            pltpu.VMEM((N, F_IN + C), jnp.bfloat16),
        ],
    )(adjs, x, x, W_l1, bl1, W_r1, W_l2, bl2, W_r2)

    hwp = jnp.zeros((NP2, C), jnp.bfloat16).at[:N].set(hw)

    def adj_b_index(i, j):
        return (i, jnp.maximum(j, (BLKB * i) // W2))

    return pl.pallas_call(
        _phase_b,
        grid=(N // BLKB, NB2),
        in_specs=[
            pl.BlockSpec((BLKB, W2), adj_b_index),
            pl.BlockSpec((NP2, C), lambda i, j: (0, 0)),
            pl.BlockSpec((BLKB, C), lambda i, j: (i, 0)),
        ],
        out_specs=pl.BlockSpec((BLKB, C), lambda i, j: (i, 0)),
        out_shape=jax.ShapeDtypeStruct((N, C), jnp.float32),
    )(adjs, hwp, acc)


# speedup vs baseline: 2.0334x; 1.0387x over previous
import jax
import jax.numpy as jnp
from jax.experimental import pallas as pl
from jax.experimental.pallas import tpu as pltpu

N, F_IN, H, C = 10000, 128, 128, 64
BLK = 400
CHUNK = 2000
BLKB = 2000
W2 = 1280
NB2 = 8
NP2 = NB2 * W2


def _phase_a(adj_ref, xf_ref, xb_ref, wl1_ref, bl1_ref, wr1_ref,
             wl2_ref, bl2_ref, wr2_ref, hw_ref, acc_ref, rhs_ref):
    r = pl.program_id(0)

    @pl.when(r == 0)
    def _():
        rhs_ref[:, :F_IN] = xf_ref[...].astype(jnp.bfloat16)
        rhs_ref[:, F_IN:] = jnp.zeros((N, C), jnp.bfloat16)

    a16 = adj_ref[...].astype(jnp.bfloat16)
    big = jnp.dot(a16, rhs_ref[...], preferred_element_type=jnp.float32)
    agg = big[:, :F_IN]
    pfx = big[:, F_IN:]

    o = jax.lax.dot_general(agg, wl1_ref[...], (((1,), (1,)), ((), ())),
                            preferred_element_type=jnp.float32)
    o = o + bl1_ref[...]
    o = o + jax.lax.dot_general(xb_ref[...], wr1_ref[...],
                                (((1,), (1,)), ((), ())),
                                preferred_element_type=jnp.float32)
    denom = jnp.maximum(jnp.sum(jnp.abs(o), axis=1, keepdims=True), 1e-12)
    h = jnp.maximum(o / denom, 0.0)

    hw_r = jax.lax.dot_general(
        h, wl2_ref[...], (((1,), (1,)), ((), ())),
        preferred_element_type=jnp.float32).astype(jnp.bfloat16)
    hw_ref[pl.ds(r * BLK, BLK), :] = hw_r
    acc_ref[pl.ds(r * BLK, BLK), :] = pfx + jax.lax.dot_general(
        h, wr2_ref[...], (((1,), (1,)), ((), ())),
        preferred_element_type=jnp.float32) + bl2_ref[...]

    @pl.when(((r + 1) * BLK) % CHUNK == 0)
    def _():
        q = (r * BLK) // CHUNK
        rhs_ref[pl.ds(q * CHUNK, CHUNK), F_IN:] = \
            hw_ref[pl.ds(q * CHUNK, CHUNK), :]


def _phase_b(adj_ref, hwp_ref, acc_ref, out_ref):
    i = pl.program_id(0)
    j = pl.program_id(1)
    jstart = (BLKB * i) // W2

    @pl.when(j == jstart)
    def _():
        out_ref[...] = acc_ref[...]

    @pl.when(j >= jstart)
    def _():
        start_off = BLKB * i - j * W2
        end_off = N - j * W2
        row = jax.lax.broadcasted_iota(jnp.int32, (W2, C), 0)
        hw_t = jnp.where((row >= start_off) & (row < end_off),
                         hwp_ref[pl.ds(j * W2, W2), :], 0)
        out_ref[...] += jnp.dot(adj_ref[...].astype(jnp.bfloat16), hw_t,
                                preferred_element_type=jnp.float32)

    @pl.when(j == NB2 - 1)
    def _():
        o = out_ref[...]
        m = jnp.max(o, axis=1, keepdims=True)
        lse = jnp.log(jnp.sum(jnp.exp(o - m), axis=1, keepdims=True))
        out_ref[...] = o - m - lse


@jax.jit
def kernel(x, adjs, W_l1, b_l1, W_r1, W_l2, b_l2, W_r2):
    nblk = N // BLK
    bl1 = b_l1.reshape(1, H)
    bl2 = b_l2.reshape(1, C)

    hw, acc = pl.pallas_call(
        _phase_a,
        grid=(nblk,),
        in_specs=[
            pl.BlockSpec((BLK, N), lambda r: (r, 0)),
            pl.BlockSpec((N, F_IN), lambda r: (0, 0)),
            pl.BlockSpec((BLK, F_IN), lambda r: (r, 0)),
            pl.BlockSpec((H, F_IN), lambda r: (0, 0)),
            pl.BlockSpec((1, H), lambda r: (0, 0)),
            pl.BlockSpec((H, F_IN), lambda r: (0, 0)),
            pl.BlockSpec((C, H), lambda r: (0, 0)),
            pl.BlockSpec((1, C), lambda r: (0, 0)),
            pl.BlockSpec((C, H), lambda r: (0, 0)),
        ],
        out_specs=[
            pl.BlockSpec((N, C), lambda r: (0, 0)),
            pl.BlockSpec((N, C), lambda r: (0, 0)),
        ],
        out_shape=[
            jax.ShapeDtypeStruct((N, C), jnp.bfloat16),
            jax.ShapeDtypeStruct((N, C), jnp.float32),
        ],
        scratch_shapes=[
            pltpu.VMEM((N, F_IN + C), jnp.bfloat16),
        ],
    )(adjs, x, x, W_l1, bl1, W_r1, W_l2, bl2, W_r2)

    hwp = jnp.zeros((NP2, C), jnp.bfloat16).at[:N].set(hw)

    def adj_b_index(i, j):
        return (i, jnp.maximum(j, (BLKB * i) // W2))

    return pl.pallas_call(
        _phase_b,
        grid=(N // BLKB, NB2),
        in_specs=[
            pl.BlockSpec((BLKB, W2), adj_b_index),
            pl.BlockSpec((NP2, C), lambda i, j: (0, 0)),
            pl.BlockSpec((BLKB, C), lambda i, j: (i, 0)),
        ],
        out_specs=pl.BlockSpec((BLKB, C), lambda i, j: (i, 0)),
        out_shape=jax.ShapeDtypeStruct((N, C), jnp.float32),
    )(adjs, hwp, acc)
